# Initial kernel scaffold; baseline (speedup 1.0000x reference)
import jax
import jax.numpy as jnp
from jax.experimental import pallas as pl


def kernel(input, indices):
    raise NotImplementedError("write your pallas kernel here")


# speedup vs baseline: 56.8115x; 56.8115x over previous
import functools

import jax
import jax.numpy as jnp
from jax import lax
from jax.experimental import pallas as pl
from jax.experimental.pallas import tpu as pltpu
from jax.experimental.pallas import tpu_sc as plsc


def kernel(input, indices):
    B, C, H, W = input.shape
    P = B * C
    Hout, Wout = 2 * H, 2 * W

    info = plsc.get_sparse_core_info()
    NC, NS = info.num_cores, info.num_subcores
    NW = NC * NS

    planes_per_w = P // NW
    RH = 48
    S = H // RH
    IN_STRIP = RH * W
    OUT_STRIP = 2 * RH * Wout

    in_flat = input.reshape(P, H * W)
    idx_flat = indices.reshape(P, H * W)

    mesh = plsc.VectorSubcoreMesh(core_axis_name="c", subcore_axis_name="s")

    @functools.partial(
        pl.kernel,
        mesh=mesh,
        out_type=jax.ShapeDtypeStruct((P, Hout * Wout), jnp.float32),
        scratch_types=[
            pltpu.VMEM((IN_STRIP,), jnp.float32),
            pltpu.VMEM((IN_STRIP,), jnp.int32),
            pltpu.VMEM((OUT_STRIP,), jnp.float32),
        ],
        compiler_params=pltpu.CompilerParams(needs_layout_passes=False),
    )
    def unpool(in_hbm, idx_hbm, out_hbm, in_v, idx_v, out_v):
        wid = lax.axis_index("s") * NC + lax.axis_index("c")
        zeros16 = jnp.zeros((16,), jnp.float32)

        def plane_body(t, _):
            p = wid * planes_per_w + t

            def strip_body(s, _):
                off = s * OUT_STRIP
                pltpu.sync_copy(in_hbm.at[p, pl.ds(s * IN_STRIP, IN_STRIP)], in_v)
                pltpu.sync_copy(idx_hbm.at[p, pl.ds(s * IN_STRIP, IN_STRIP)], idx_v)

                def zero_body(j, _):
                    out_v[pl.ds(j * 16, 16)] = zeros16
                    return 0

                lax.fori_loop(0, OUT_STRIP // 16, zero_body, 0)

                def scat_body(i, _):
                    vals = in_v[pl.ds(i * 16, 16)]
                    ids = idx_v[pl.ds(i * 16, 16)] - off
                    plsc.store_scatter(out_v, [ids], vals)
                    return 0

                lax.fori_loop(0, IN_STRIP // 16, scat_body, 0)
                pltpu.sync_copy(out_v, out_hbm.at[p, pl.ds(off, OUT_STRIP)])
                return 0

            lax.fori_loop(0, S, strip_body, 0)
            return 0

        lax.fori_loop(0, planes_per_w, plane_body, 0)

    out = unpool(in_flat, idx_flat)
    return out.reshape(B, C, Hout, Wout)


# speedup vs baseline: 100.2190x; 1.7641x over previous
import functools

import jax
import jax.numpy as jnp
from jax import lax
from jax.experimental import pallas as pl
from jax.experimental.pallas import tpu as pltpu
from jax.experimental.pallas import tpu_sc as plsc


def kernel(input, indices):
    B, C, H, W = input.shape
    P = B * C
    Hout, Wout = 2 * H, 2 * W

    info = plsc.get_sparse_core_info()
    NC, NS = info.num_cores, info.num_subcores
    NW = NC * NS

    planes_per_w = P // NW
    RH = 48
    S = H // RH
    IN_STRIP = RH * W
    OUT_STRIP = 2 * RH * Wout
    NSTRIPS = planes_per_w * S

    in_flat = input.reshape(P * H * W)
    idx_flat = indices.reshape(P * H * W)

    mesh = plsc.VectorSubcoreMesh(core_axis_name="c", subcore_axis_name="s")

    @functools.partial(
        pl.kernel,
        mesh=mesh,
        out_type=jax.ShapeDtypeStruct((P * Hout * Wout,), jnp.float32),
        scratch_types=[
            pltpu.VMEM((IN_STRIP,), jnp.float32),
            pltpu.VMEM((IN_STRIP,), jnp.float32),
            pltpu.VMEM((IN_STRIP,), jnp.int32),
            pltpu.VMEM((IN_STRIP,), jnp.int32),
            pltpu.VMEM((OUT_STRIP,), jnp.float32),
            pltpu.VMEM((OUT_STRIP,), jnp.float32),
            pltpu.SemaphoreType.DMA,
            pltpu.SemaphoreType.DMA,
            pltpu.SemaphoreType.DMA,
            pltpu.SemaphoreType.DMA,
            pltpu.SemaphoreType.DMA,
            pltpu.SemaphoreType.DMA,
        ],
        compiler_params=pltpu.CompilerParams(needs_layout_passes=False),
    )
    def unpool(in_hbm, idx_hbm, out_hbm, in_v0, in_v1, idx_v0, idx_v1,
               out_v0, out_v1, si0, si1, sx0, sx1, so0, so1):
        wid = lax.axis_index("s") * NC + lax.axis_index("c")
        base = wid * NSTRIPS
        in_b = (in_v0, in_v1)
        idx_b = (idx_v0, idx_v1)
        out_b = (out_v0, out_v1)
        sin = (si0, si1)
        sidx = (sx0, sx1)
        sout = (so0, so1)
        zeros16 = jnp.zeros((16,), jnp.float32)

        def in_copy(ke, b):
            g = base + ke
            return pltpu.make_async_copy(
                in_hbm.at[pl.ds(g * IN_STRIP, IN_STRIP)], in_b[b], sin[b])

        def idx_copy(ke, b):
            g = base + ke
            return pltpu.make_async_copy(
                idx_hbm.at[pl.ds(g * IN_STRIP, IN_STRIP)], idx_b[b], sidx[b])

        def out_copy(ke, b):
            g = base + ke
            return pltpu.make_async_copy(
                out_b[b], out_hbm.at[pl.ds(g * OUT_STRIP, OUT_STRIP)], sout[b])

        in_copy(0, 0).start()
        idx_copy(0, 0).start()
        in_copy(1, 1).start()
        idx_copy(1, 1).start()

        def pair_body(kk, _):
            k = kk * 2
            for b in range(2):
                ke = k + b
                in_copy(ke, b).wait()
                idx_copy(ke, b).wait()

                @pl.when(ke >= 2)
                def _():
                    out_copy(ke - 2, b).wait()

                def zero_body(j, _):
                    out_b[b][pl.ds(j * 16, 16)] = zeros16
                    return 0

                lax.fori_loop(0, OUT_STRIP // 16, zero_body, 0, unroll=8)

                off = (ke % S) * OUT_STRIP

                def scat_body(i, _):
                    vals = in_b[b][pl.ds(i * 16, 16)]
                    ids = idx_b[b][pl.ds(i * 16, 16)] - off
                    plsc.store_scatter(out_b[b], [ids], vals)
                    return 0

                lax.fori_loop(0, IN_STRIP // 16, scat_body, 0, unroll=8)

                out_copy(ke, b).start()

                @pl.when(ke + 2 < NSTRIPS)
                def _():
                    in_copy(ke + 2, b).start()
                    idx_copy(ke + 2, b).start()
            return 0

        lax.fori_loop(0, NSTRIPS // 2, pair_body, 0)
        out_copy(NSTRIPS - 2, 0).wait()
        out_copy(NSTRIPS - 1, 1).wait()

    out = unpool(in_flat, idx_flat)
    return out.reshape(B, C, Hout, Wout)
